# baseline (speedup 1.0000x reference)
import jax
import jax.numpy as jnp
from jax import lax
from jax.experimental import pallas as pl
from jax.experimental.pallas import tpu as pltpu
from jax.experimental.pallas import tpu_sc as plsc

_NOISE_ROWS = 64
_ROW = 160000
_NROWS = 2 * _NOISE_ROWS
_NC = 2
_NS = 16
_NW = _NC * _NS
_ROWS_PER_W = _NROWS // _NW
_CHUNK = 32000
_SPM_CHUNK = 16000
_SPM_FLOATS = 64000
_NBUF_T = 3
_NBUF_S = 2

_PERM = (
    22, 18, 6, 26, 21, 45, 60, 39, 61, 49, 38, 27, 32, 57, 10, 63,
    35, 20, 24, 56, 52, 40, 51, 42, 55, 4, 31, 14, 0, 43, 34, 3,
    50, 5, 17, 37, 28, 2, 41, 23, 58, 44, 54, 48, 46, 36, 1, 8,
    16, 33, 30, 7, 19, 15, 9, 62, 13, 11, 59, 47, 25, 53, 12, 29,
)

_ROWMAP = _PERM + tuple(range(_NOISE_ROWS, _NROWS))


class _Pipe:
    def __init__(self, n_steps, nbuf, in_copy, out_copy):
        self.n = n_steps
        self.nbuf = nbuf
        self.in_copy = in_copy
        self.out_copy = out_copy
        self.ins = {}
        self.outs = {}
        self.look = max(1, nbuf - 1)

    def prime(self):
        for k in range(min(self.look, self.n)):
            self.ins[k] = self.in_copy(k, k % self.nbuf)

    def step(self, t):
        if t >= self.n:
            return
        self.ins[t].wait()
        self.outs[t] = self.out_copy(t, t % self.nbuf)
        nt = t + self.look
        if nt < self.n:
            prev = nt - self.nbuf
            if prev >= 0:
                self.outs[prev].wait()
            self.ins[nt] = self.in_copy(nt, nt % self.nbuf)

    def drain(self):
        for t in range(max(0, self.n - self.nbuf), self.n):
            self.outs[t].wait()


def _remix_body(src, out, tbuf, spm, sem_ti, sem_to, sem_si, sem_so):
    cid = lax.axis_index("c")
    sid = lax.axis_index("s")
    wid = sid * _NC + cid

    srows = []
    for i in range(_ROWS_PER_W):
        m = jnp.int32(_ROWMAP[i])
        for w in range(1, _NW):
            m = jnp.where(wid == w, _ROWMAP[w * _ROWS_PER_W + i], m)
        srows.append(m)
    orow0 = wid * _ROWS_PER_W

    tile_steps = []
    spm_steps = []
    for i in range(_ROWS_PER_W):
        for off in range(_SPM_FLOATS, _ROW, _CHUNK):
            tile_steps.append((i, off))
        for off in range(0, _SPM_FLOATS, _SPM_CHUNK):
            spm_steps.append((i, off))

    def offs(step):
        i, c = step
        off_in = pl.multiple_of(srows[i] * _ROW + c, 8)
        off_out = pl.multiple_of((orow0 + i) * _ROW + c, 8)
        return off_in, off_out

    def tile_in(t, slot):
        off_in, _ = offs(tile_steps[t])
        return pltpu.async_copy(
            src.at[pl.ds(off_in, _CHUNK)],
            tbuf.at[pl.ds(slot * _CHUNK, _CHUNK)],
            sem_ti,
        )

    def tile_out(t, slot):
        _, off_out = offs(tile_steps[t])
        return pltpu.async_copy(
            tbuf.at[pl.ds(slot * _CHUNK, _CHUNK)],
            out.at[pl.ds(off_out, _CHUNK)],
            sem_to,
        )

    spm_base = sid * _NBUF_S * _SPM_CHUNK

    def spm_in(t, slot):
        off_in, _ = offs(spm_steps[t])
        dst = pl.multiple_of(spm_base + slot * _SPM_CHUNK, 8)
        return pltpu.async_copy(
            src.at[pl.ds(off_in, _SPM_CHUNK)], spm.at[pl.ds(dst, _SPM_CHUNK)], sem_si
        )

    def spm_out(t, slot):
        _, off_out = offs(spm_steps[t])
        s = pl.multiple_of(spm_base + slot * _SPM_CHUNK, 8)
        return pltpu.async_copy(
            spm.at[pl.ds(s, _SPM_CHUNK)], out.at[pl.ds(off_out, _SPM_CHUNK)], sem_so
        )

    tp = _Pipe(len(tile_steps), _NBUF_T, tile_in, tile_out)
    sp = _Pipe(len(spm_steps), _NBUF_S, spm_in, spm_out)
    tp.prime()
    sp.prime()
    for k in range(max(tp.n, sp.n)):
        tp.step(k)
        sp.step(k)
    tp.drain()
    sp.drain()


def kernel(sources):
    src = sources.reshape(_NROWS * _ROW)
    out = pl.kernel(
        _remix_body,
        out_type=jax.ShapeDtypeStruct((_NROWS * _ROW,), jnp.float32),
        mesh=plsc.VectorSubcoreMesh(core_axis_name="c", subcore_axis_name="s"),
        scratch_types=[
            pltpu.VMEM((_NBUF_T * _CHUNK,), jnp.float32),
            pltpu.VMEM_SHARED((_NS * _NBUF_S * _SPM_CHUNK,), jnp.float32),
            pltpu.SemaphoreType.DMA,
            pltpu.SemaphoreType.DMA,
            pltpu.SemaphoreType.DMA,
            pltpu.SemaphoreType.DMA,
        ],
        name="sc_remix_copy",
    )(src)
    return out.reshape(2, _NOISE_ROWS, 1, _ROW)


# speedup vs baseline: 1.0081x; 1.0081x over previous
import jax
import jax.numpy as jnp
from jax import lax
from jax.experimental import pallas as pl
from jax.experimental.pallas import tpu as pltpu
from jax.experimental.pallas import tpu_sc as plsc

_NOISE_ROWS = 64
_ROW = 160000
_NROWS = 2 * _NOISE_ROWS
_NC = 2
_NS = 16
_NW = _NC * _NS
_ROWS_PER_W = _NROWS // _NW
_CHUNK = 40000
_CHUNKS_PER_ROW = _ROW // _CHUNK
_STEPS = _ROWS_PER_W * _CHUNKS_PER_ROW
_NBUF = 3

_PERM = (
    22, 18, 6, 26, 21, 45, 60, 39, 61, 49, 38, 27, 32, 57, 10, 63,
    35, 20, 24, 56, 52, 40, 51, 42, 55, 4, 31, 14, 0, 43, 34, 3,
    50, 5, 17, 37, 28, 2, 41, 23, 58, 44, 54, 48, 46, 36, 1, 8,
    16, 33, 30, 7, 19, 15, 9, 62, 13, 11, 59, 47, 25, 53, 12, 29,
)

_ROWMAP = _PERM + tuple(range(_NOISE_ROWS, _NROWS))


def _remix_body(src, out, buf, sem_in, sem_out):
    cid = lax.axis_index("c")
    sid = lax.axis_index("s")
    wid = sid * _NC + cid

    srows = []
    for i in range(_ROWS_PER_W):
        m = jnp.int32(_ROWMAP[i])
        for w in range(1, _NW):
            m = jnp.where(wid == w, _ROWMAP[w * _ROWS_PER_W + i], m)
        srows.append(m)
    orow0 = wid * _ROWS_PER_W

    def in_copy(t, slot):
        i, c = divmod(t, _CHUNKS_PER_ROW)
        off = pl.multiple_of(srows[i] * _ROW + c * _CHUNK, 8)
        return pltpu.async_copy(
            src.at[pl.ds(off, _CHUNK)], buf.at[pl.ds(slot * _CHUNK, _CHUNK)], sem_in
        )

    def out_copy(t, slot):
        i, c = divmod(t, _CHUNKS_PER_ROW)
        off = pl.multiple_of((orow0 + i) * _ROW + c * _CHUNK, 8)
        return pltpu.async_copy(
            buf.at[pl.ds(slot * _CHUNK, _CHUNK)], out.at[pl.ds(off, _CHUNK)], sem_out
        )

    ins, outs = {}, {}
    ins[0] = in_copy(0, 0)
    ins[1] = in_copy(1, 1)
    for t in range(_STEPS):
        ins[t].wait()
        outs[t] = out_copy(t, t % _NBUF)
        if t + 2 < _STEPS:
            if t - 1 >= 0:
                outs[t - 1].wait()
            ins[t + 2] = in_copy(t + 2, (t + 2) % _NBUF)
    for t in range(_STEPS - 3, _STEPS):
        outs[t].wait()


def kernel(sources):
    src = sources.reshape(_NROWS * _ROW)
    out = pl.kernel(
        _remix_body,
        out_type=jax.ShapeDtypeStruct((_NROWS * _ROW,), jnp.float32),
        mesh=plsc.VectorSubcoreMesh(core_axis_name="c", subcore_axis_name="s"),
        scratch_types=[
            pltpu.VMEM((_NBUF * _CHUNK,), jnp.float32),
            pltpu.SemaphoreType.DMA,
            pltpu.SemaphoreType.DMA,
        ],
        name="sc_remix_copy",
    )(src)
    return out.reshape(2, _NOISE_ROWS, 1, _ROW)
